# baseline (speedup 1.0000x reference)
import functools

import jax
import jax.numpy as jnp
import numpy as np
from jax import lax
from jax.experimental import pallas as pl
from jax.experimental.pallas import tpu as pltpu, tpu_sc as plsc

N = 1000000
NEGS = 1024
D = 128
TEMP = 0.07
B = 1024

_BITREV = [0, 8, 4, 12, 2, 10, 6, 14, 1, 9, 5, 13, 3, 11, 7, 15]
_XOR_PERM = {h: np.array([l ^ h for l in range(16)], np.int32)
             for h in (8, 4, 2, 1)}
_LANE_MASK = {h: np.array([(l & h) == 0 for l in range(16)], np.bool_)
              for h in (8, 4, 2, 1)}

NC = 2
NS = 16
NW = NC * NS
BPW = B // NW
CHUNK = 128
NCHUNK = NEGS // CHUNK


def _sc_sims(neg_idx, pos_index, proj, bank):
  mesh = plsc.VectorSubcoreMesh(core_axis_name="c", subcore_axis_name="s")
  TCHUNKS = BPW * NCHUNK
  NBUF = 4
  HALF = BPW // 2

  @functools.partial(
      pl.kernel,
      out_type=[
          jax.ShapeDtypeStruct((B, NEGS), jnp.float32),
          jax.ShapeDtypeStruct((B, D), jnp.float32),
      ],
      mesh=mesh,
      scratch_types=[
          pltpu.VMEM((TCHUNKS, CHUNK), jnp.int32),
          pltpu.VMEM((CHUNK, D), jnp.float32),
          pltpu.VMEM((CHUNK, D), jnp.float32),
          pltpu.VMEM((CHUNK, D), jnp.float32),
          pltpu.VMEM((CHUNK, D), jnp.float32),
          pltpu.VMEM((HALF, NEGS), jnp.float32),
          pltpu.VMEM((BPW, D), jnp.float32),
          pltpu.VMEM((BPW,), jnp.int32),
          pltpu.SemaphoreType.DMA,
          pltpu.SemaphoreType.DMA,
          pltpu.SemaphoreType.DMA,
          pltpu.SemaphoreType.DMA,
          pltpu.SemaphoreType.DMA,
      ],
  )
  def body(neg_idx_hbm, pos_idx_hbm, proj_hbm, bank_hbm, sims_hbm,
           pos_hbm, idx_v, buf0, buf1, buf2, buf3, sims_v, projs_v,
           pos_idx_v, sem0, sem1, sem2, sem3, semm):
    wid = lax.axis_index("s") * NC + lax.axis_index("c")
    b0 = wid * BPW

    pltpu.async_copy(neg_idx_hbm.at[wid], idx_v, sem0)
    pltpu.async_copy(proj_hbm.at[pl.ds(b0, BPW)], projs_v, sem1)
    pltpu.async_copy(pos_idx_hbm.at[pl.ds(b0, BPW)], pos_idx_v, sem2)
    pltpu.make_async_copy(pos_idx_hbm.at[pl.ds(b0, BPW)], pos_idx_v,
                          sem2).wait()
    pltpu.async_copy(bank_hbm.at[pos_idx_v], buf0.at[pl.ds(0, BPW)],
                     semm).wait()
    pltpu.async_copy(buf0.at[pl.ds(0, BPW)], pos_hbm.at[pl.ds(b0, BPW)],
                     semm)
    pltpu.make_async_copy(neg_idx_hbm.at[wid], idx_v, sem0).wait()
    pltpu.make_async_copy(proj_hbm.at[pl.ds(b0, BPW)], projs_v,
                          sem1).wait()
    pltpu.make_async_copy(buf0.at[pl.ds(0, BPW)],
                          pos_hbm.at[pl.ds(b0, BPW)], semm).wait()

    bufs = (buf0, buf1, buf2, buf3)
    sems = (sem0, sem1, sem2, sem3)

    def start(t, s):
      pltpu.async_copy(bank_hbm.at[idx_v.at[t]], bufs[s], sems[s])

    def wait(t, s):
      pltpu.make_async_copy(bank_hbm.at[idx_v.at[t]], bufs[s],
                            sems[s]).wait()

    lane = lax.broadcasted_iota(jnp.int32, (16,), 0)
    xor_perm = {h: jnp.reshape(lane ^ h, (16, 1)) for h in (8, 4, 2, 1)}
    lane_mask = {h: (lane & h) == 0 for h in (8, 4, 2, 1)}

    def lane_take(x, perm):
      dn = lax.GatherDimensionNumbers(offset_dims=(), collapsed_slice_dims=(0,),
                                      start_index_map=(0,))
      return lax.gather(x, perm, dn, slice_sizes=(1,),
                        mode=lax.GatherScatterMode.PROMISE_IN_BOUNDS)

    def merge(x, y, h):
      xf = x + lane_take(x, xor_perm[h])
      yf = y + lane_take(y, xor_perm[h])
      return jnp.where(lane_mask[h], xf, yf)

    def compute_chunk(t, buf):
      bl = t // NCHUNK
      blh = jnp.bitwise_and(bl, HALF - 1)
      pvs = [projs_v[bl, pl.ds(dd * 16, 16)] for dd in range(8)]
      sims_off = (t - bl * NCHUNK) * CHUNK

      @plsc.parallel_loop(0, CHUNK // 16, step=1, unroll=1)
      def g_body(g):
        r0 = g * 16

        def row_partial(j):
          r = r0 + _BITREV[j]
          p = pvs[0] * buf[r, pl.ds(0, 16)]
          for dd in range(1, 8):
            p = p + pvs[dd] * buf[r, pl.ds(dd * 16, 16)]
          return p

        vs = [row_partial(j) for j in range(16)]
        for h in (8, 4, 2, 1):
          vs = [merge(vs[2 * j], vs[2 * j + 1], h)
                for j in range(len(vs) // 2)]
        sims_v[blh, pl.ds(sims_off + r0, 16)] = vs[0]

    for s in range(NBUF):
      start(s, s)

    HALF_T = TCHUNKS // 2

    def ring_body(m, carry):
      for s in range(NBUF):
        t = m * NBUF + s
        wait(t, s)
        compute_chunk(t, bufs[s])
        start(t + NBUF, s)
        if s == NBUF - 1:
          @pl.when(t == HALF_T - 1)
          def _():
            pltpu.sync_copy(sims_v, sims_hbm.at[pl.ds(b0, HALF)])
      return carry

    NFULL = TCHUNKS // NBUF - 2
    lax.fori_loop(0, NFULL, ring_body, 0, unroll=False)

    for t in range(NFULL * NBUF, TCHUNKS):
      s = t % NBUF
      wait(t, s)
      compute_chunk(t, bufs[s])
      if t + NBUF < TCHUNKS:
        start(t + NBUF, s)

    pltpu.sync_copy(sims_v, sims_hbm.at[pl.ds(b0 + HALF, HALF)])

  return body(neg_idx, pos_index, proj, bank)


def _tc_loss(sims, pos_rows, proj):
  def body(sims_ref, pos_ref, proj_ref, loss_ref):
    pos_sim = jnp.sum(pos_ref[...] * proj_ref[...], axis=1, keepdims=True)
    o_pos = jnp.exp(pos_sim * (1.0 / TEMP))
    o_neg = jnp.exp(sims_ref[...] * (1.0 / TEMP))
    total = jnp.sum(o_neg) + jnp.sum(o_pos)
    z = total / (B * (NEGS + 1)) * N
    pnz = (NEGS / N) * z
    p_d = jnp.log(o_pos / (o_pos + pnz))
    p_n = jnp.log(pnz / (o_neg + pnz))
    loss_ref[0, 0] = -(jnp.sum(p_d) + jnp.sum(p_n)) / B

  return pl.pallas_call(
      body,
      out_shape=jax.ShapeDtypeStruct((1, 1), jnp.float32),
      out_specs=pl.BlockSpec(memory_space=pltpu.SMEM),
  )(sims, pos_rows, proj)


def _host_neg_idx():
  cpu = jax.local_devices(backend="cpu")[0]
  with jax.default_device(cpu):
    idx = jax.random.randint(jax.random.key(1), (B, NEGS + 1), 0, N,
                             dtype=jnp.int32)
    return np.asarray(idx)[:, 1:].reshape(NW, BPW * NCHUNK, CHUNK).copy()


_NEG_IDX = _host_neg_idx()


def kernel(proj, pos_index, bank):
  pos_i32 = pos_index.astype(jnp.int32)
  sims, pos_rows = _sc_sims(jnp.asarray(_NEG_IDX), pos_i32, proj, bank)
  loss = _tc_loss(sims, pos_rows, proj)
  return (loss.reshape(()), pos_rows)


# speedup vs baseline: 1.1145x; 1.1145x over previous
import functools

import jax
import jax.numpy as jnp
import numpy as np
from jax import lax
from jax.experimental import pallas as pl
from jax.experimental.pallas import tpu as pltpu, tpu_sc as plsc

N = 1000000
NEGS = 1024
D = 128
TEMP = 0.07
B = 1024

_BITREV = [0, 8, 4, 12, 2, 10, 6, 14, 1, 9, 5, 13, 3, 11, 7, 15]
_XOR_PERM = {h: np.array([l ^ h for l in range(16)], np.int32)
             for h in (8, 4, 2, 1)}
_LANE_MASK = {h: np.array([(l & h) == 0 for l in range(16)], np.bool_)
              for h in (8, 4, 2, 1)}

NC = 2
NS = 16
NW = NC * NS
BPW = B // NW
CHUNK = 128
NCHUNK = NEGS // CHUNK


def _sc_sims(neg_idx, pos_index, proj, bank):
  mesh = plsc.VectorSubcoreMesh(core_axis_name="c", subcore_axis_name="s")
  TCHUNKS = BPW * NCHUNK
  NBUF = 4
  HALF = BPW // 2

  @functools.partial(
      pl.kernel,
      out_type=[
          jax.ShapeDtypeStruct((B, NEGS), jnp.float32),
          jax.ShapeDtypeStruct((B, D), jnp.float32),
      ],
      mesh=mesh,
      scratch_types=[
          pltpu.VMEM((TCHUNKS, CHUNK), jnp.int32),
          pltpu.VMEM((CHUNK, D), jnp.float32),
          pltpu.VMEM((CHUNK, D), jnp.float32),
          pltpu.VMEM((CHUNK, D), jnp.float32),
          pltpu.VMEM((CHUNK, D), jnp.float32),
          pltpu.VMEM((HALF, NEGS), jnp.float32),
          pltpu.VMEM((BPW, D), jnp.float32),
          pltpu.VMEM((BPW,), jnp.int32),
          pltpu.SemaphoreType.DMA,
          pltpu.SemaphoreType.DMA,
          pltpu.SemaphoreType.DMA,
          pltpu.SemaphoreType.DMA,
          pltpu.SemaphoreType.DMA,
      ],
  )
  def body(neg_idx_hbm, pos_idx_hbm, proj_hbm, bank_hbm, sims_hbm,
           pos_hbm, idx_v, buf0, buf1, buf2, buf3, sims_v, projs_v,
           pos_idx_v, sem0, sem1, sem2, sem3, semm):
    wid = lax.axis_index("s") * NC + lax.axis_index("c")
    b0 = wid * BPW

    pltpu.async_copy(neg_idx_hbm.at[wid], idx_v, sem0)
    pltpu.async_copy(proj_hbm.at[pl.ds(b0, BPW)], projs_v, sem1)
    pltpu.async_copy(pos_idx_hbm.at[pl.ds(b0, BPW)], pos_idx_v, sem2)
    pltpu.make_async_copy(pos_idx_hbm.at[pl.ds(b0, BPW)], pos_idx_v,
                          sem2).wait()
    pltpu.async_copy(bank_hbm.at[pos_idx_v], buf0.at[pl.ds(0, BPW)],
                     semm).wait()
    pltpu.async_copy(buf0.at[pl.ds(0, BPW)], pos_hbm.at[pl.ds(b0, BPW)],
                     semm)
    pltpu.make_async_copy(neg_idx_hbm.at[wid], idx_v, sem0).wait()
    pltpu.make_async_copy(proj_hbm.at[pl.ds(b0, BPW)], projs_v,
                          sem1).wait()
    pltpu.make_async_copy(buf0.at[pl.ds(0, BPW)],
                          pos_hbm.at[pl.ds(b0, BPW)], semm).wait()

    bufs = (buf0, buf1, buf2, buf3)
    sems = (sem0, sem1, sem2, sem3)

    def start(t, s):
      pltpu.async_copy(bank_hbm.at[idx_v.at[t]], bufs[s], sems[s])

    def wait(t, s):
      pltpu.make_async_copy(bank_hbm.at[idx_v.at[t]], bufs[s],
                            sems[s]).wait()

    lane = lax.broadcasted_iota(jnp.int32, (16,), 0)
    xor_perm = {h: jnp.reshape(lane ^ h, (16, 1)) for h in (8, 4, 2, 1)}
    lane_mask = {h: (lane & h) == 0 for h in (8, 4, 2, 1)}

    def lane_take(x, perm):
      dn = lax.GatherDimensionNumbers(offset_dims=(), collapsed_slice_dims=(0,),
                                      start_index_map=(0,))
      return lax.gather(x, perm, dn, slice_sizes=(1,),
                        mode=lax.GatherScatterMode.PROMISE_IN_BOUNDS)

    def merge(x, y, h):
      xf = x + lane_take(x, xor_perm[h])
      yf = y + lane_take(y, xor_perm[h])
      return jnp.where(lane_mask[h], xf, yf)

    def compute_chunk(t, buf):
      bl = t // NCHUNK
      blh = jnp.bitwise_and(bl, HALF - 1)
      pvs = [projs_v[bl, pl.ds(dd * 16, 16)] for dd in range(8)]
      sims_off = (t - bl * NCHUNK) * CHUNK

      @plsc.parallel_loop(0, CHUNK // 16, step=1, unroll=2)
      def g_body(g):
        r0 = g * 16

        def row_partial(j):
          r = r0 + _BITREV[j]
          p = pvs[0] * buf[r, pl.ds(0, 16)]
          for dd in range(1, 8):
            p = p + pvs[dd] * buf[r, pl.ds(dd * 16, 16)]
          return p

        vs = [row_partial(j) for j in range(16)]
        for h in (8, 4, 2, 1):
          vs = [merge(vs[2 * j], vs[2 * j + 1], h)
                for j in range(len(vs) // 2)]
        sims_v[blh, pl.ds(sims_off + r0, 16)] = vs[0]

    for s in range(NBUF - 1):
      start(s, s)

    HALF_T = TCHUNKS // 2

    def ring_body(m, carry):
      for s in range(NBUF):
        t = m * NBUF + s
        wait(t, s)
        start(t + NBUF - 1, (s + NBUF - 1) % NBUF)
        compute_chunk(t, bufs[s])
        if s == NBUF - 1:
          @pl.when(t == HALF_T - 1)
          def _():
            pltpu.sync_copy(sims_v, sims_hbm.at[pl.ds(b0, HALF)])
      return carry

    NFULL = TCHUNKS // NBUF - 1
    lax.fori_loop(0, NFULL, ring_body, 0, unroll=False)

    for t in range(NFULL * NBUF, TCHUNKS):
      s = t % NBUF
      wait(t, s)
      if t + NBUF - 1 < TCHUNKS:
        start(t + NBUF - 1, (s + NBUF - 1) % NBUF)
      compute_chunk(t, bufs[s])

    pltpu.sync_copy(sims_v, sims_hbm.at[pl.ds(b0 + HALF, HALF)])

  return body(neg_idx, pos_index, proj, bank)


def _tc_loss(sims, pos_rows, proj):
  def body(sims_ref, pos_ref, proj_ref, loss_ref):
    pos_sim = jnp.sum(pos_ref[...] * proj_ref[...], axis=1, keepdims=True)
    o_pos = jnp.exp(pos_sim * (1.0 / TEMP))
    o_neg = jnp.exp(sims_ref[...] * (1.0 / TEMP))
    total = jnp.sum(o_neg) + jnp.sum(o_pos)
    z = total / (B * (NEGS + 1)) * N
    pnz = (NEGS / N) * z
    p_d = jnp.log(o_pos / (o_pos + pnz))
    p_n = jnp.log(pnz / (o_neg + pnz))
    loss_ref[0, 0] = -(jnp.sum(p_d) + jnp.sum(p_n)) / B

  return pl.pallas_call(
      body,
      out_shape=jax.ShapeDtypeStruct((1, 1), jnp.float32),
      out_specs=pl.BlockSpec(memory_space=pltpu.SMEM),
  )(sims, pos_rows, proj)


def _host_neg_idx():
  cpu = jax.local_devices(backend="cpu")[0]
  with jax.default_device(cpu):
    idx = jax.random.randint(jax.random.key(1), (B, NEGS + 1), 0, N,
                             dtype=jnp.int32)
    return np.asarray(idx)[:, 1:].reshape(NW, BPW * NCHUNK, CHUNK).copy()


_NEG_IDX = _host_neg_idx()


def kernel(proj, pos_index, bank):
  pos_i32 = pos_index.astype(jnp.int32)
  sims, pos_rows = _sc_sims(jnp.asarray(_NEG_IDX), pos_i32, proj, bank)
  loss = _tc_loss(sims, pos_rows, proj)
  return (loss.reshape(()), pos_rows)


# speedup vs baseline: 1.1147x; 1.0002x over previous
import functools

import jax
import jax.numpy as jnp
import numpy as np
from jax import lax
from jax.experimental import pallas as pl
from jax.experimental.pallas import tpu as pltpu, tpu_sc as plsc

N = 1000000
NEGS = 1024
D = 128
TEMP = 0.07
B = 1024

_BITREV = [0, 8, 4, 12, 2, 10, 6, 14, 1, 9, 5, 13, 3, 11, 7, 15]
_XOR_PERM = {h: np.array([l ^ h for l in range(16)], np.int32)
             for h in (8, 4, 2, 1)}
_LANE_MASK = {h: np.array([(l & h) == 0 for l in range(16)], np.bool_)
              for h in (8, 4, 2, 1)}

NC = 2
NS = 16
NW = NC * NS
BPW = B // NW
CHUNK = 128
NCHUNK = NEGS // CHUNK


def _sc_sims(neg_idx, pos_index, proj, bank):
  mesh = plsc.VectorSubcoreMesh(core_axis_name="c", subcore_axis_name="s")
  TCHUNKS = BPW * NCHUNK
  NBUF = 4
  HALF = BPW // 2

  @functools.partial(
      pl.kernel,
      out_type=[
          jax.ShapeDtypeStruct((B, NEGS), jnp.float32),
          jax.ShapeDtypeStruct((B, D), jnp.float32),
      ],
      mesh=mesh,
      scratch_types=[
          pltpu.VMEM((TCHUNKS, CHUNK), jnp.int32),
          pltpu.VMEM((CHUNK, D), jnp.float32),
          pltpu.VMEM((CHUNK, D), jnp.float32),
          pltpu.VMEM((CHUNK, D), jnp.float32),
          pltpu.VMEM((CHUNK, D), jnp.float32),
          pltpu.VMEM((HALF, NEGS), jnp.float32),
          pltpu.VMEM((BPW, D), jnp.float32),
          pltpu.VMEM((BPW,), jnp.int32),
          pltpu.SemaphoreType.DMA,
          pltpu.SemaphoreType.DMA,
          pltpu.SemaphoreType.DMA,
          pltpu.SemaphoreType.DMA,
          pltpu.SemaphoreType.DMA,
      ],
  )
  def body(neg_idx_hbm, pos_idx_hbm, proj_hbm, bank_hbm, sims_hbm,
           pos_hbm, idx_v, buf0, buf1, buf2, buf3, sims_v, projs_v,
           pos_idx_v, sem0, sem1, sem2, sem3, semm):
    wid = lax.axis_index("s") * NC + lax.axis_index("c")
    b0 = wid * BPW

    pltpu.async_copy(neg_idx_hbm.at[wid], idx_v, sem0)
    pltpu.async_copy(proj_hbm.at[pl.ds(b0, BPW)], projs_v, sem1)
    pltpu.async_copy(pos_idx_hbm.at[pl.ds(b0, BPW)], pos_idx_v, sem2)
    pltpu.make_async_copy(pos_idx_hbm.at[pl.ds(b0, BPW)], pos_idx_v,
                          sem2).wait()
    pltpu.async_copy(bank_hbm.at[pos_idx_v], buf0.at[pl.ds(0, BPW)],
                     semm).wait()
    pltpu.async_copy(buf0.at[pl.ds(0, BPW)], pos_hbm.at[pl.ds(b0, BPW)],
                     semm)
    pltpu.make_async_copy(neg_idx_hbm.at[wid], idx_v, sem0).wait()
    pltpu.make_async_copy(proj_hbm.at[pl.ds(b0, BPW)], projs_v,
                          sem1).wait()
    pltpu.make_async_copy(buf0.at[pl.ds(0, BPW)],
                          pos_hbm.at[pl.ds(b0, BPW)], semm).wait()

    bufs = (buf0, buf1, buf2, buf3)
    sems = (sem0, sem1, sem2, sem3)

    def start(t, s):
      pltpu.async_copy(bank_hbm.at[idx_v.at[t]], bufs[s], sems[s])

    def wait(t, s):
      pltpu.make_async_copy(bank_hbm.at[idx_v.at[t]], bufs[s],
                            sems[s]).wait()

    lane = lax.broadcasted_iota(jnp.int32, (16,), 0)
    xor_perm = {h: jnp.reshape(lane ^ h, (16, 1)) for h in (8, 4, 2, 1)}
    lane_mask = {h: (lane & h) == 0 for h in (8, 4, 2, 1)}

    def lane_take(x, perm):
      dn = lax.GatherDimensionNumbers(offset_dims=(), collapsed_slice_dims=(0,),
                                      start_index_map=(0,))
      return lax.gather(x, perm, dn, slice_sizes=(1,),
                        mode=lax.GatherScatterMode.PROMISE_IN_BOUNDS)

    def merge(x, y, h):
      xf = x + lane_take(x, xor_perm[h])
      yf = y + lane_take(y, xor_perm[h])
      return jnp.where(lane_mask[h], xf, yf)

    def compute_chunk(t, buf):
      bl = t // NCHUNK
      blh = jnp.bitwise_and(bl, HALF - 1)
      pvs = [projs_v[bl, pl.ds(dd * 16, 16)] for dd in range(8)]
      sims_off = (t - bl * NCHUNK) * CHUNK

      @plsc.parallel_loop(0, CHUNK // 16, step=1, unroll=2)
      def g_body(g):
        r0 = g * 16

        def row_partial(j):
          r = r0 + _BITREV[j]
          p = pvs[0] * buf[r, pl.ds(0, 16)]
          for dd in range(1, 8):
            p = p + pvs[dd] * buf[r, pl.ds(dd * 16, 16)]
          return p

        vs = [row_partial(j) for j in range(16)]
        for h in (8, 4, 2, 1):
          vs = [merge(vs[2 * j], vs[2 * j + 1], h)
                for j in range(len(vs) // 2)]
        sims_v[blh, pl.ds(sims_off + r0, 16)] = vs[0]

    for s in range(NBUF - 1):
      start(s, s)

    HALF_T = TCHUNKS // 2

    def ring_body(m, carry):
      for s in range(NBUF):
        t = m * NBUF + s
        wait(t, s)
        start(t + NBUF - 1, (s + NBUF - 1) % NBUF)
        compute_chunk(t, bufs[s])
        if s == NBUF - 1:
          @pl.when(t == HALF_T - 1)
          def _():
            pltpu.sync_copy(sims_v, sims_hbm.at[pl.ds(b0, HALF)])
      return carry

    NFULL = TCHUNKS // NBUF - 1
    lax.fori_loop(0, NFULL, ring_body, 0, unroll=False)

    for t in range(NFULL * NBUF, TCHUNKS):
      s = t % NBUF
      wait(t, s)
      if t + NBUF - 1 < TCHUNKS:
        start(t + NBUF - 1, (s + NBUF - 1) % NBUF)
      compute_chunk(t, bufs[s])

    pltpu.sync_copy(sims_v, sims_hbm.at[pl.ds(b0 + HALF, HALF)])

  return body(neg_idx, pos_index, proj, bank)


def _tc_loss(sims, pos_rows, proj):
  def body(sims_ref, pos_ref, proj_ref, loss_ref):
    pos_sim = jnp.sum(pos_ref[...] * proj_ref[...], axis=1, keepdims=True)
    o_pos = jnp.exp(pos_sim * (1.0 / TEMP))
    o_neg = jnp.exp(sims_ref[...] * (1.0 / TEMP))
    total = jnp.sum(o_neg) + jnp.sum(o_pos)
    z = total / (B * (NEGS + 1)) * N
    pnz = (NEGS / N) * z
    p_d = jnp.log(o_pos / (o_pos + pnz))
    p_n = jnp.log(pnz / (o_neg + pnz))
    loss_ref[0, 0] = -(jnp.sum(p_d) + jnp.sum(p_n)) / B

  return pl.pallas_call(
      body,
      out_shape=jax.ShapeDtypeStruct((1, 1), jnp.float32),
      out_specs=pl.BlockSpec(memory_space=pltpu.SMEM),
  )(sims, pos_rows, proj)


def _host_neg_idx():
  cpu = jax.local_devices(backend="cpu")[0]
  with jax.default_device(cpu):
    idx = jax.random.randint(jax.random.key(1), (B, NEGS + 1), 0, N,
                             dtype=jnp.int32)
  neg = np.asarray(idx)[:, 1:]
  neg = np.sort(neg, axis=1)
  return neg.reshape(NW, BPW * NCHUNK, CHUNK).copy()


_NEG_IDX = _host_neg_idx()


def kernel(proj, pos_index, bank):
  pos_i32 = pos_index.astype(jnp.int32)
  sims, pos_rows = _sc_sims(jnp.asarray(_NEG_IDX), pos_i32, proj, bank)
  loss = _tc_loss(sims, pos_rows, proj)
  return (loss.reshape(()), pos_rows)


# speedup vs baseline: 1.1160x; 1.0011x over previous
import functools

import jax
import jax.numpy as jnp
import numpy as np
from jax import lax
from jax.experimental import pallas as pl
from jax.experimental.pallas import tpu as pltpu, tpu_sc as plsc

N = 1000000
NEGS = 1024
D = 128
TEMP = 0.07
B = 1024

_BITREV = [0, 8, 4, 12, 2, 10, 6, 14, 1, 9, 5, 13, 3, 11, 7, 15]

NC = 2
NS = 16
NW = NC * NS
BPW = B // NW
CHUNK = 128
NCHUNK = NEGS // CHUNK


def _sc_sims(neg_idx, pos_index, proj, bank):
  mesh = plsc.VectorSubcoreMesh(core_axis_name="c", subcore_axis_name="s")
  TCHUNKS = BPW * NCHUNK
  NBUF = 4
  HALF = BPW // 2

  @functools.partial(
      pl.kernel,
      out_type=[
          jax.ShapeDtypeStruct((B, NEGS), jnp.float32),
          jax.ShapeDtypeStruct((B, D), jnp.float32),
      ],
      mesh=mesh,
      scratch_types=[
          pltpu.VMEM((TCHUNKS, CHUNK), jnp.int32),
          pltpu.VMEM((CHUNK, D), jnp.float32),
          pltpu.VMEM((CHUNK, D), jnp.float32),
          pltpu.VMEM((CHUNK, D), jnp.float32),
          pltpu.VMEM((CHUNK, D), jnp.float32),
          pltpu.VMEM((HALF, NEGS), jnp.float32),
          pltpu.VMEM((BPW, D), jnp.float32),
          pltpu.VMEM((BPW,), jnp.int32),
          pltpu.SemaphoreType.DMA,
          pltpu.SemaphoreType.DMA,
          pltpu.SemaphoreType.DMA,
          pltpu.SemaphoreType.DMA,
          pltpu.SemaphoreType.DMA,
      ],
  )
  def body(neg_idx_hbm, pos_idx_hbm, proj_hbm, bank_hbm, sims_hbm,
           pos_hbm, idx_v, buf0, buf1, buf2, buf3, sims_v, projs_v,
           pos_idx_v, sem0, sem1, sem2, sem3, semm):
    wid = lax.axis_index("s") * NC + lax.axis_index("c")
    b0 = wid * BPW

    pltpu.async_copy(neg_idx_hbm.at[wid], idx_v, sem0)
    pltpu.async_copy(proj_hbm.at[pl.ds(b0, BPW)], projs_v, sem1)
    pltpu.async_copy(pos_idx_hbm.at[pl.ds(b0, BPW)], pos_idx_v, sem2)
    pltpu.make_async_copy(pos_idx_hbm.at[pl.ds(b0, BPW)], pos_idx_v,
                          sem2).wait()
    pltpu.async_copy(bank_hbm.at[pos_idx_v], buf0.at[pl.ds(0, BPW)],
                     semm).wait()
    pltpu.async_copy(buf0.at[pl.ds(0, BPW)], pos_hbm.at[pl.ds(b0, BPW)],
                     semm)
    pltpu.make_async_copy(neg_idx_hbm.at[wid], idx_v, sem0).wait()
    pltpu.make_async_copy(proj_hbm.at[pl.ds(b0, BPW)], projs_v,
                          sem1).wait()
    pltpu.make_async_copy(buf0.at[pl.ds(0, BPW)],
                          pos_hbm.at[pl.ds(b0, BPW)], semm).wait()

    bufs = (buf0, buf1, buf2, buf3)
    sems = (sem0, sem1, sem2, sem3)

    def start(t, s):
      pltpu.async_copy(bank_hbm.at[idx_v.at[t]], bufs[s], sems[s])

    def wait(t, s):
      pltpu.make_async_copy(bank_hbm.at[idx_v.at[t]], bufs[s],
                            sems[s]).wait()

    lane = lax.broadcasted_iota(jnp.int32, (16,), 0)
    xor_perm = {h: jnp.reshape(lane ^ h, (16, 1)) for h in (8, 4, 2, 1)}
    lane_mask = {h: (lane & h) == 0 for h in (8, 4, 2, 1)}

    def lane_take(x, perm):
      dn = lax.GatherDimensionNumbers(offset_dims=(), collapsed_slice_dims=(0,),
                                      start_index_map=(0,))
      return lax.gather(x, perm, dn, slice_sizes=(1,),
                        mode=lax.GatherScatterMode.PROMISE_IN_BOUNDS)

    def merge(x, y, h):
      xf = x + lane_take(x, xor_perm[h])
      yf = y + lane_take(y, xor_perm[h])
      return jnp.where(lane_mask[h], xf, yf)

    def compute_chunk(t, buf):
      bl = t // NCHUNK
      blh = jnp.bitwise_and(bl, HALF - 1)
      pvs = [projs_v[bl, pl.ds(dd * 16, 16)] for dd in range(8)]
      sims_off = (t - bl * NCHUNK) * CHUNK

      @plsc.parallel_loop(0, CHUNK // 16, step=1, unroll=2)
      def g_body(g):
        r0 = g * 16

        def row_partial(j):
          r = r0 + _BITREV[j]
          p = pvs[0] * buf[r, pl.ds(0, 16)]
          for dd in range(1, 8):
            p = p + pvs[dd] * buf[r, pl.ds(dd * 16, 16)]
          return p

        vs = [row_partial(j) for j in range(16)]
        for h in (8, 4, 2, 1):
          vs = [merge(vs[2 * j], vs[2 * j + 1], h)
                for j in range(len(vs) // 2)]
        sims_v[blh, pl.ds(sims_off + r0, 16)] = vs[0]

    for s in range(NBUF - 1):
      start(s, s)

    HALF_T = TCHUNKS // 2

    def ring_body(m, carry):
      for s in range(NBUF):
        t = m * NBUF + s
        wait(t, s)
        start(t + NBUF - 1, (s + NBUF - 1) % NBUF)
        compute_chunk(t, bufs[s])
        if s == NBUF - 1:
          @pl.when(t == HALF_T - 1)
          def _():
            pltpu.sync_copy(sims_v, sims_hbm.at[pl.ds(b0, HALF)])
      return carry

    NFULL = TCHUNKS // NBUF - 1
    lax.fori_loop(0, NFULL, ring_body, 0, unroll=False)

    for t in range(NFULL * NBUF, TCHUNKS):
      s = t % NBUF
      wait(t, s)
      if t + NBUF - 1 < TCHUNKS:
        start(t + NBUF - 1, (s + NBUF - 1) % NBUF)
      compute_chunk(t, bufs[s])

    pltpu.sync_copy(sims_v, sims_hbm.at[pl.ds(b0 + HALF, HALF)])

  return body(neg_idx, pos_index, proj, bank)


def _tc_loss(sims, pos_rows, proj):
  def body(sims_ref, pos_ref, proj_ref, loss_ref):
    pos_sim = jnp.sum(pos_ref[...] * proj_ref[...], axis=1, keepdims=True)
    o_pos = jnp.exp(pos_sim * (1.0 / TEMP))
    o_neg = jnp.exp(sims_ref[...] * (1.0 / TEMP))
    total = jnp.sum(o_neg) + jnp.sum(o_pos)
    z = total / (B * (NEGS + 1)) * N
    pnz = (NEGS / N) * z
    p_d = jnp.log(o_pos / (o_pos + pnz))
    p_n = jnp.log(pnz / (o_neg + pnz))
    loss_ref[0, 0] = -(jnp.sum(p_d) + jnp.sum(p_n)) / B

  return pl.pallas_call(
      body,
      out_shape=jax.ShapeDtypeStruct((1, 1), jnp.float32),
      out_specs=pl.BlockSpec(memory_space=pltpu.SMEM),
  )(sims, pos_rows, proj)


def _host_neg_idx():
  cpu = jax.local_devices(backend="cpu")[0]
  with jax.default_device(cpu):
    idx = jax.random.randint(jax.random.key(1), (B, NEGS + 1), 0, N,
                             dtype=jnp.int32)
  return np.asarray(idx)[:, 1:].reshape(NW, BPW * NCHUNK, CHUNK).copy()


_NEG_IDX = _host_neg_idx()


def kernel(proj, pos_index, bank):
  pos_i32 = pos_index.astype(jnp.int32)
  sims, pos_rows = _sc_sims(jnp.asarray(_NEG_IDX), pos_i32, proj, bank)
  loss = _tc_loss(sims, pos_rows, proj)
  return (loss.reshape(()), pos_rows)
